# baseline (speedup 1.0000x reference)
import jax
import jax.numpy as jnp
from jax import lax
from jax.experimental import pallas as pl
from jax.experimental.pallas import tpu as pltpu

_RB = 512


def _body(idx_ref, noise_ref, c_ref, mns_hbm, out_ref, rows, sem):
    i = pl.program_id(0)

    def issue(j, _):
        row = idx_ref[i * _RB + j]
        pltpu.make_async_copy(
            mns_hbm.at[pl.ds(row, 1), :],
            rows.at[pl.ds(j, 1), :],
            sem,
        ).start()
        return 0

    lax.fori_loop(0, _RB, issue, 0, unroll=8)

    pltpu.make_async_copy(mns_hbm.at[pl.ds(0, _RB), :], rows, sem).wait()

    y = lax.dot_general(
        noise_ref[...], c_ref[...],
        dimension_numbers=(((1,), (1,)), ((), ())),
        preferred_element_type=jnp.float32,
    )
    out_ref[...] = y + rows[...]


def kernel(inds, noise, mns, c):
    B, D = noise.shape
    idx = inds.astype(jnp.int32)
    grid = B // _RB

    return pl.pallas_call(
        _body,
        grid_spec=pltpu.PrefetchScalarGridSpec(
            num_scalar_prefetch=1,
            grid=(grid,),
            in_specs=[
                pl.BlockSpec((_RB, D), lambda i, idx_ref: (i, 0)),
                pl.BlockSpec((D, D), lambda i, idx_ref: (0, 0)),
                pl.BlockSpec(memory_space=pltpu.MemorySpace.HBM),
            ],
            out_specs=pl.BlockSpec((_RB, D), lambda i, idx_ref: (i, 0)),
            scratch_shapes=[
                pltpu.VMEM((_RB, D), jnp.float32),
                pltpu.SemaphoreType.DMA,
            ],
        ),
        out_shape=jax.ShapeDtypeStruct((B, D), jnp.float32),
    )(idx, noise, c, mns)


# speedup vs baseline: 1.0058x; 1.0058x over previous
import jax
import jax.numpy as jnp
from jax import lax
from jax.experimental import pallas as pl
from jax.experimental.pallas import tpu as pltpu

_RB = 512


def _body(idx_ref, noise_ref, c_ref, mns_hbm, out_ref, rows, sem0, sem1, sem2, sem3):
    i = pl.program_id(0)
    sems = (sem0, sem1, sem2, sem3)
    q = _RB // 4

    for s in range(4):
        def issue(j, _, s=s):
            row = idx_ref[i * _RB + s * q + j]
            pltpu.make_async_copy(
                mns_hbm.at[pl.ds(row, 1), :],
                rows.at[pl.ds(s * q + j, 1), :],
                sems[s],
            ).start()
            return 0

        lax.fori_loop(0, q, issue, 0, unroll=8)

    for s in range(4):
        pltpu.make_async_copy(
            mns_hbm.at[pl.ds(0, q), :],
            rows.at[pl.ds(s * q, q), :],
            sems[s],
        ).wait()

    y = lax.dot_general(
        noise_ref[...], c_ref[...],
        dimension_numbers=(((1,), (1,)), ((), ())),
        preferred_element_type=jnp.float32,
    )
    out_ref[...] = y + rows[...]


def kernel(inds, noise, mns, c):
    B, D = noise.shape
    idx = inds.astype(jnp.int32)
    grid = B // _RB

    return pl.pallas_call(
        _body,
        grid_spec=pltpu.PrefetchScalarGridSpec(
            num_scalar_prefetch=1,
            grid=(grid,),
            in_specs=[
                pl.BlockSpec((_RB, D), lambda i, idx_ref: (i, 0)),
                pl.BlockSpec((D, D), lambda i, idx_ref: (0, 0)),
                pl.BlockSpec(memory_space=pltpu.MemorySpace.HBM),
            ],
            out_specs=pl.BlockSpec((_RB, D), lambda i, idx_ref: (i, 0)),
            scratch_shapes=[
                pltpu.VMEM((_RB, D), jnp.float32),
                pltpu.SemaphoreType.DMA,
                pltpu.SemaphoreType.DMA,
                pltpu.SemaphoreType.DMA,
                pltpu.SemaphoreType.DMA,
            ],
        ),
        out_shape=jax.ShapeDtypeStruct((B, D), jnp.float32),
    )(idx, noise, c, mns)
